# baseline (speedup 1.0000x reference)
import functools
import math

import jax
import jax.numpy as jnp
import numpy as np
from jax import lax
from jax.experimental import pallas as pl
from jax.experimental.pallas import tpu as pltpu
from jax.experimental.pallas import tpu_sc as plsc

B = 16
N_SNPS = 100000
N_NODES = 300000
N_GENES = 18000
NF = 8
DH = 64
FD = 16
MAIN_DIM = 15

NC = 2
NS = 16
NW = NC * NS
CK = 128
NPH = 5
PH = 16
NCHUNK = NPH * PH
NPAD = NW * NCHUNK * CK
ACC_ROWS = 18432
ZROWS = ACC_ROWS // NS

GP = 640
NGC = 5
TGENES = GP * NGC
SPAD = NW * TGENES

_BN_SCALE = 1.0 / math.sqrt(1.0 + 1e-5)


@functools.partial(
    pl.kernel,
    out_type=jax.ShapeDtypeStruct((SPAD * B,), jnp.float32),
    mesh=plsc.VectorSubcoreMesh(core_axis_name="c", subcore_axis_name="s"),
    scratch_types=[
        pltpu.VMEM((2, B, GP), jnp.float32),
        pltpu.VMEM((2, NF, GP), jnp.float32),
        pltpu.VMEM((GP * B,), jnp.float32),
        pltpu.VMEM((GP * B,), jnp.float32),
        pltpu.SemaphoreType.DMA,
        pltpu.SemaphoreType.DMA,
    ],
    compiler_params=pltpu.CompilerParams(needs_layout_passes=False),
)
def _sc_build(snp_hbm, filt_hbm, tab_hbm, sbuf, fbuf, obuf0, obuf1, lsem, wsem):
    c = lax.axis_index("c")
    s = lax.axis_index("s")
    wid = s * NC + c
    lane = jax.lax.iota(jnp.int32, 16)

    def _src(k):
        gbase = wid * TGENES + k * GP
        return (snp_hbm.at[:, pl.ds(gbase, GP)],
                filt_hbm.at[:, pl.ds(gbase, GP)])

    def _fire_load(k):
        sp, fl = _src(k)
        pltpu.async_copy(sp, sbuf.at[k % 2], lsem)
        pltpu.async_copy(fl, fbuf.at[k % 2], lsem)

    def _wait_load(k):
        sp, fl = _src(k)
        pltpu.make_async_copy(sp, sbuf.at[k % 2], lsem).wait()
        pltpu.make_async_copy(fl, fbuf.at[k % 2], lsem).wait()

    def _tab_slice(k):
        return tab_hbm.at[pl.ds((wid * TGENES + k * GP) * B, GP * B)]

    _fire_load(0)
    for k in range(NGC):
        _wait_load(k)
        if k + 1 < NGC:
            _fire_load(k + 1)
        if k >= 2:
            pltpu.make_async_copy(obuf0 if k % 2 == 0 else obuf1,
                                  _tab_slice(k - 2), wsem).wait()

        sb = sbuf.at[k % 2]
        fb = fbuf.at[k % 2]
        ob = obuf0 if k % 2 == 0 else obuf1

        @plsc.parallel_loop(0, GP // 16, unroll=4)
        def _group(q):
            go = q * 16
            f = fb[0, pl.ds(go, 16)]
            for t in range(1, NF):
                f = f + fb[t, pl.ds(go, 16)]
            f = f * (1.0 / NF)
            base = go * B + lane * B
            for b in range(B):
                v = sb[b, pl.ds(go, 16)] * f
                plsc.store_scatter(ob, [base + b], v)
        pltpu.async_copy(ob, _tab_slice(k), wsem)
    pltpu.make_async_copy(obuf0 if (NGC - 2) % 2 == 0 else obuf1,
                          _tab_slice(NGC - 2), wsem).wait()
    pltpu.make_async_copy(obuf0 if (NGC - 1) % 2 == 0 else obuf1,
                          _tab_slice(NGC - 1), wsem).wait()


@functools.partial(
    pl.kernel,
    out_type=jax.ShapeDtypeStruct((NC * ACC_ROWS, B), jnp.float32),
    mesh=plsc.VectorSubcoreMesh(core_axis_name="c", subcore_axis_name="s"),
    scratch_types=[
        pltpu.VMEM((NCHUNK, CK), jnp.int32),
        pltpu.VMEM((NCHUNK, CK), jnp.int32),
        pltpu.VMEM((2, PH, CK, B), jnp.float32),
        pltpu.VMEM_SHARED((ACC_ROWS, B), jnp.float32),
        pltpu.SemaphoreType.DMA,
        pltpu.SemaphoreType.DMA,
    ],
    compiler_params=pltpu.CompilerParams(use_tc_tiling_on_sc=False),
)
def _sc_segsum(tab_hbm, ids_hbm, segs_hbm, out_hbm,
               idx_v, seg_v, rows_v, acc, gsem, ssem):
    c = lax.axis_index("c")
    s = lax.axis_index("s")
    wid = s * NC + c

    def _zrow(i, _):
        rows_v[0, 0, i, :] = jnp.zeros((B,), jnp.float32)
        return 0
    lax.fori_loop(0, CK, _zrow, 0)

    def _zacc(k, _):
        pltpu.sync_copy(rows_v.at[0, 0], acc.at[pl.ds(s * ZROWS + k * CK, CK)])
        return 0
    lax.fori_loop(0, ZROWS // CK, _zacc, 0)

    pltpu.sync_copy(ids_hbm.at[wid], idx_v)
    pltpu.sync_copy(segs_hbm.at[wid], seg_v)

    plsc.subcore_barrier()

    def _fire_g(ph, st):
        def f(r, _):
            pltpu.async_copy(tab_hbm.at[idx_v.at[ph * PH + r]],
                             rows_v.at[st, r], gsem)
            return 0
        lax.fori_loop(0, PH, f, 0)

    def _drain_g(ph, st):
        def f(r, _):
            pltpu.make_async_copy(tab_hbm.at[idx_v.at[ph * PH + r]],
                                  rows_v.at[st, r], gsem).wait()
            return 0
        lax.fori_loop(0, PH, f, 0)

    def _fire_s(ph, st):
        def f(r, _):
            pltpu.async_copy(rows_v.at[st, r],
                             acc.at[seg_v.at[ph * PH + r]], ssem, add=True)
            return 0
        lax.fori_loop(0, PH, f, 0)

    def _drain_s(ph, st):
        def f(r, _):
            pltpu.make_async_copy(rows_v.at[st, r],
                                  acc.at[seg_v.at[ph * PH + r]], ssem).wait()
            return 0
        lax.fori_loop(0, PH, f, 0)

    _fire_g(0, 0)
    for ph in range(NPH):
        st = ph % 2
        _drain_g(ph, st)
        if ph > 0:
            _drain_s(ph - 1, 1 - st)
        _fire_s(ph, st)
        if ph + 1 < NPH:
            _fire_g(ph + 1, 1 - st)
    _drain_s(NPH - 1, (NPH - 1) % 2)

    plsc.subcore_barrier()
    pltpu.sync_copy(acc.at[pl.ds(s * ZROWS, ZROWS)],
                    out_hbm.at[pl.ds(c * ACC_ROWS + s * ZROWS, ZROWS)])


_QROWS = ACC_ROWS // 8


def _mlp_body(parts_ref, W1r_ref, b1_ref, g1_ref, be1_ref,
              W2_ref, b2_ref, g2_ref, be2_ref, Wm_ref, bm_ref, out_ref):
    p = parts_ref[0:_QROWS, :] + parts_ref[_QROWS:2 * _QROWS, :]
    h = jnp.zeros((DH, B), jnp.float32)
    for j in range(8):
        h = h + lax.dot_general(W1r_ref[j], p[:, j * B:(j + 1) * B],
                                (((1,), (0,)), ((), ())),
                                preferred_element_type=jnp.float32)
    h = h + b1_ref[...]
    h = h * (g1_ref[...] * _BN_SCALE) + be1_ref[...]
    h = jnp.maximum(h, 0.0)
    h = lax.dot_general(W2_ref[...], h, (((1,), (0,)), ((), ())),
                        preferred_element_type=jnp.float32)
    h = h + b2_ref[...]
    h = h * (g2_ref[...] * _BN_SCALE) + be2_ref[...]
    h = jnp.maximum(h, 0.0)
    out_ref[...] = lax.dot_general(h, Wm_ref[...], (((0,), (1,)), ((), ())),
                                   preferred_element_type=jnp.float32) + bm_ref[...]


_mlp = pl.pallas_call(
    _mlp_body,
    out_shape=jax.ShapeDtypeStruct((B, 1), jnp.float32),
)


def kernel(snp, snp_ids, segment_ids, filters,
           W1, b1, bn1_w, bn1_b, W2, b2, bn2_w, bn2_b, Wm, bm):
    pad = NPAD - N_NODES
    pad_pos = np.arange(pad, dtype=np.int32)
    pad_idx = jnp.asarray(pad_pos % N_SNPS)
    pad_seg = jnp.asarray(N_GENES + (pad_pos % (ACC_ROWS - N_GENES)))
    ids3 = jnp.concatenate([snp_ids, pad_idx]).reshape(NW, NCHUNK, CK)
    segs3 = jnp.concatenate([segment_ids, pad_seg]).reshape(NW, NCHUNK, CK)
    Wm_p = jnp.pad(Wm, ((0, 0), (0, FD - MAIN_DIM)))
    W1r = jnp.pad(W1.reshape(DH, N_GENES // 8, 8).transpose(2, 0, 1),
                  ((0, 0), (0, 0), (0, (ACC_ROWS - N_GENES) // 8)))
    snp_p = jnp.pad(snp, ((0, 0), (0, SPAD - N_SNPS)))
    filt_p = jnp.pad(filters, ((0, 0), (0, SPAD - N_SNPS)))
    tab_flat = _sc_build(snp_p, filt_p)
    tab = tab_flat.reshape(SPAD, B)
    parts = _sc_segsum(tab, ids3, segs3)
    p2 = parts.reshape(NC * ACC_ROWS * B // 128, 128)
    return _mlp(p2, W1r,
                b1.reshape(DH, 1), bn1_w.reshape(DH, 1), bn1_b.reshape(DH, 1),
                W2, b2.reshape(FD, 1), bn2_w.reshape(FD, 1), bn2_b.reshape(FD, 1),
                Wm_p, bm.reshape(1, 1))


# speedup vs baseline: 1.0520x; 1.0520x over previous
import functools
import math

import jax
import jax.numpy as jnp
import numpy as np
from jax import lax
from jax.experimental import pallas as pl
from jax.experimental.pallas import tpu as pltpu
from jax.experimental.pallas import tpu_sc as plsc

B = 16
N_SNPS = 100000
N_NODES = 300000
N_GENES = 18000
NF = 8
DH = 64
FD = 16
MAIN_DIM = 15

NC = 2
NS = 16
NW = NC * NS
CK = 128
NPH = 5
PH = 16
NCHUNK = NPH * PH
NPAD = NW * NCHUNK * CK
ACC_ROWS = 18432
ZROWS = ACC_ROWS // NS

GP = 640
NGC = 5
TGENES = GP * NGC
SPAD = NW * TGENES

_BN_SCALE = 1.0 / math.sqrt(1.0 + 1e-5)


@functools.partial(
    pl.kernel,
    out_type=jax.ShapeDtypeStruct((SPAD * B,), jnp.float32),
    mesh=plsc.VectorSubcoreMesh(core_axis_name="c", subcore_axis_name="s"),
    scratch_types=[
        pltpu.VMEM((2, B, GP), jnp.float32),
        pltpu.VMEM((2, NF, GP), jnp.float32),
        pltpu.VMEM((GP * B,), jnp.float32),
        pltpu.VMEM((GP * B,), jnp.float32),
        pltpu.SemaphoreType.DMA,
        pltpu.SemaphoreType.DMA,
    ],
    compiler_params=pltpu.CompilerParams(needs_layout_passes=False),
)
def _sc_build(snp_hbm, filt_hbm, tab_hbm, sbuf, fbuf, obuf0, obuf1, lsem, wsem):
    c = lax.axis_index("c")
    s = lax.axis_index("s")
    wid = s * NC + c
    lane = jax.lax.iota(jnp.int32, 16)

    def _src(k):
        gbase = wid * TGENES + k * GP
        return (snp_hbm.at[:, pl.ds(gbase, GP)],
                filt_hbm.at[:, pl.ds(gbase, GP)])

    def _fire_load(k):
        sp, fl = _src(k)
        pltpu.async_copy(sp, sbuf.at[k % 2], lsem)
        pltpu.async_copy(fl, fbuf.at[k % 2], lsem)

    def _wait_load(k):
        sp, fl = _src(k)
        pltpu.make_async_copy(sp, sbuf.at[k % 2], lsem).wait()
        pltpu.make_async_copy(fl, fbuf.at[k % 2], lsem).wait()

    def _tab_slice(k):
        return tab_hbm.at[pl.ds((wid * TGENES + k * GP) * B, GP * B)]

    _fire_load(0)
    for k in range(NGC):
        _wait_load(k)
        if k + 1 < NGC:
            _fire_load(k + 1)
        if k >= 2:
            pltpu.make_async_copy(obuf0 if k % 2 == 0 else obuf1,
                                  _tab_slice(k - 2), wsem).wait()

        sb = sbuf.at[k % 2]
        fb = fbuf.at[k % 2]
        ob = obuf0 if k % 2 == 0 else obuf1

        @plsc.parallel_loop(0, GP // 16, unroll=2)
        def _group(q):
            go = q * 16
            f = fb[0, pl.ds(go, 16)]
            for t in range(1, NF):
                f = f + fb[t, pl.ds(go, 16)]
            f = f * (1.0 / NF)
            base = go * B + lane * B
            for b in range(B):
                v = sb[b, pl.ds(go, 16)] * f
                plsc.store_scatter(ob, [base + b], v)
        pltpu.async_copy(ob, _tab_slice(k), wsem)
    pltpu.make_async_copy(obuf0 if (NGC - 2) % 2 == 0 else obuf1,
                          _tab_slice(NGC - 2), wsem).wait()
    pltpu.make_async_copy(obuf0 if (NGC - 1) % 2 == 0 else obuf1,
                          _tab_slice(NGC - 1), wsem).wait()


@functools.partial(
    pl.kernel,
    out_type=jax.ShapeDtypeStruct((NC * ACC_ROWS, B), jnp.float32),
    mesh=plsc.VectorSubcoreMesh(core_axis_name="c", subcore_axis_name="s"),
    scratch_types=[
        pltpu.VMEM((NCHUNK, CK), jnp.int32),
        pltpu.VMEM((NCHUNK, CK), jnp.int32),
        pltpu.VMEM((2, PH, CK, B), jnp.float32),
        pltpu.VMEM_SHARED((ACC_ROWS, B), jnp.float32),
        pltpu.SemaphoreType.DMA,
        pltpu.SemaphoreType.DMA,
    ],
    compiler_params=pltpu.CompilerParams(use_tc_tiling_on_sc=False),
)
def _sc_segsum(tab_hbm, ids_hbm, segs_hbm, out_hbm,
               idx_v, seg_v, rows_v, acc, gsem, ssem):
    c = lax.axis_index("c")
    s = lax.axis_index("s")
    wid = s * NC + c

    def _zrow(i, _):
        rows_v[0, 0, i, :] = jnp.zeros((B,), jnp.float32)
        return 0
    lax.fori_loop(0, CK, _zrow, 0)

    def _zacc(k, _):
        pltpu.sync_copy(rows_v.at[0, 0], acc.at[pl.ds(s * ZROWS + k * CK, CK)])
        return 0
    lax.fori_loop(0, ZROWS // CK, _zacc, 0)

    pltpu.sync_copy(ids_hbm.at[wid], idx_v)
    pltpu.sync_copy(segs_hbm.at[wid], seg_v)

    plsc.subcore_barrier()

    def _fire_g(ph, st):
        def f(r, _):
            pltpu.async_copy(tab_hbm.at[idx_v.at[ph * PH + r]],
                             rows_v.at[st, r], gsem)
            return 0
        lax.fori_loop(0, PH, f, 0)

    def _drain_g(ph, st):
        def f(r, _):
            pltpu.make_async_copy(tab_hbm.at[idx_v.at[ph * PH + r]],
                                  rows_v.at[st, r], gsem).wait()
            return 0
        lax.fori_loop(0, PH, f, 0)

    def _fire_s(ph, st):
        def f(r, _):
            pltpu.async_copy(rows_v.at[st, r],
                             acc.at[seg_v.at[ph * PH + r]], ssem, add=True)
            return 0
        lax.fori_loop(0, PH, f, 0)

    def _drain_s(ph, st):
        def f(r, _):
            pltpu.make_async_copy(rows_v.at[st, r],
                                  acc.at[seg_v.at[ph * PH + r]], ssem).wait()
            return 0
        lax.fori_loop(0, PH, f, 0)

    _fire_g(0, 0)
    for ph in range(NPH):
        st = ph % 2
        _drain_g(ph, st)
        if ph > 0:
            _drain_s(ph - 1, 1 - st)
        _fire_s(ph, st)
        if ph + 1 < NPH:
            _fire_g(ph + 1, 1 - st)
    _drain_s(NPH - 1, (NPH - 1) % 2)

    plsc.subcore_barrier()
    pltpu.sync_copy(acc.at[pl.ds(s * ZROWS, ZROWS)],
                    out_hbm.at[pl.ds(c * ACC_ROWS + s * ZROWS, ZROWS)])


_QROWS = ACC_ROWS // 8


def _mlp_body(parts_ref, W1r_ref, b1_ref, g1_ref, be1_ref,
              W2_ref, b2_ref, g2_ref, be2_ref, Wm_ref, bm_ref, out_ref):
    p = parts_ref[0:_QROWS, :] + parts_ref[_QROWS:2 * _QROWS, :]
    h = jnp.zeros((DH, B), jnp.float32)
    for j in range(8):
        h = h + lax.dot_general(W1r_ref[j], p[:, j * B:(j + 1) * B],
                                (((1,), (0,)), ((), ())),
                                preferred_element_type=jnp.float32)
    h = h + b1_ref[...]
    h = h * (g1_ref[...] * _BN_SCALE) + be1_ref[...]
    h = jnp.maximum(h, 0.0)
    h = lax.dot_general(W2_ref[...], h, (((1,), (0,)), ((), ())),
                        preferred_element_type=jnp.float32)
    h = h + b2_ref[...]
    h = h * (g2_ref[...] * _BN_SCALE) + be2_ref[...]
    h = jnp.maximum(h, 0.0)
    out_ref[...] = lax.dot_general(h, Wm_ref[...], (((0,), (1,)), ((), ())),
                                   preferred_element_type=jnp.float32) + bm_ref[...]


_mlp = pl.pallas_call(
    _mlp_body,
    out_shape=jax.ShapeDtypeStruct((B, 1), jnp.float32),
)


def kernel(snp, snp_ids, segment_ids, filters,
           W1, b1, bn1_w, bn1_b, W2, b2, bn2_w, bn2_b, Wm, bm):
    pad = NPAD - N_NODES
    pad_pos = np.arange(pad, dtype=np.int32)
    pad_idx = jnp.asarray(pad_pos % N_SNPS)
    pad_seg = jnp.asarray(N_GENES + (pad_pos % (ACC_ROWS - N_GENES)))
    ids3 = jnp.concatenate([snp_ids, pad_idx]).reshape(NW, NCHUNK, CK)
    segs3 = jnp.concatenate([segment_ids, pad_seg]).reshape(NW, NCHUNK, CK)
    Wm_p = jnp.pad(Wm, ((0, 0), (0, FD - MAIN_DIM)))
    W1r = jnp.pad(W1.reshape(DH, N_GENES // 8, 8).transpose(2, 0, 1),
                  ((0, 0), (0, 0), (0, (ACC_ROWS - N_GENES) // 8)))
    snp_p = jnp.pad(snp, ((0, 0), (0, SPAD - N_SNPS)))
    filt_p = jnp.pad(filters, ((0, 0), (0, SPAD - N_SNPS)))
    tab_flat = _sc_build(snp_p, filt_p)
    tab = tab_flat.reshape(SPAD, B)
    parts = _sc_segsum(tab, ids3, segs3)
    p2 = parts.reshape(NC * ACC_ROWS * B // 128, 128)
    return _mlp(p2, W1r,
                b1.reshape(DH, 1), bn1_w.reshape(DH, 1), bn1_b.reshape(DH, 1),
                W2, b2.reshape(FD, 1), bn2_w.reshape(FD, 1), bn2_b.reshape(FD, 1),
                Wm_p, bm.reshape(1, 1))


# speedup vs baseline: 1.0803x; 1.0269x over previous
import functools
import math

import jax
import jax.numpy as jnp
import numpy as np
from jax import lax
from jax.experimental import pallas as pl
from jax.experimental.pallas import tpu as pltpu
from jax.experimental.pallas import tpu_sc as plsc

B = 16
N_SNPS = 100000
N_NODES = 300000
N_GENES = 18000
NF = 8
DH = 64
FD = 16
MAIN_DIM = 15

NC = 2
NS = 16
NW = NC * NS
CK = 128
NPH = 5
PH = 16
NCHUNK = NPH * PH
NPAD = NW * NCHUNK * CK
ACC_ROWS = 18432
ZROWS = ACC_ROWS // NS

GP = 640
NGC = 5
TGENES = GP * NGC
SPAD = NW * TGENES

_BN_SCALE = 1.0 / math.sqrt(1.0 + 1e-5)


@functools.partial(
    pl.kernel,
    out_type=jax.ShapeDtypeStruct((SPAD * B,), jnp.float32),
    mesh=plsc.VectorSubcoreMesh(core_axis_name="c", subcore_axis_name="s"),
    scratch_types=[
        pltpu.VMEM((2, B, GP), jnp.float32),
        pltpu.VMEM((2, NF, GP), jnp.float32),
        pltpu.VMEM((GP * B,), jnp.float32),
        pltpu.VMEM((GP * B,), jnp.float32),
        pltpu.SemaphoreType.DMA,
        pltpu.SemaphoreType.DMA,
    ],
    compiler_params=pltpu.CompilerParams(needs_layout_passes=False),
)
def _sc_build(snp_hbm, filt_hbm, tab_hbm, sbuf, fbuf, obuf0, obuf1, lsem, wsem):
    c = lax.axis_index("c")
    s = lax.axis_index("s")
    wid = s * NC + c
    lane = jax.lax.iota(jnp.int32, 16)

    def _src(k):
        gbase = wid * TGENES + k * GP
        return (snp_hbm.at[:, pl.ds(gbase, GP)],
                filt_hbm.at[:, pl.ds(gbase, GP)])

    def _fire_load(k):
        sp, fl = _src(k)
        pltpu.async_copy(sp, sbuf.at[k % 2], lsem)
        pltpu.async_copy(fl, fbuf.at[k % 2], lsem)

    def _wait_load(k):
        sp, fl = _src(k)
        pltpu.make_async_copy(sp, sbuf.at[k % 2], lsem).wait()
        pltpu.make_async_copy(fl, fbuf.at[k % 2], lsem).wait()

    def _tab_slice(k):
        return tab_hbm.at[pl.ds((wid * TGENES + k * GP) * B, GP * B)]

    _fire_load(0)
    for k in range(NGC):
        _wait_load(k)
        if k + 1 < NGC:
            _fire_load(k + 1)
        if k >= 2:
            pltpu.make_async_copy(obuf0 if k % 2 == 0 else obuf1,
                                  _tab_slice(k - 2), wsem).wait()

        sb = sbuf.at[k % 2]
        fb = fbuf.at[k % 2]
        ob = obuf0 if k % 2 == 0 else obuf1

        @plsc.parallel_loop(0, GP // 16, unroll=2)
        def _group(q):
            go = q * 16
            f = fb[0, pl.ds(go, 16)]
            for t in range(1, NF):
                f = f + fb[t, pl.ds(go, 16)]
            f = f * (1.0 / NF)
            base = go * B + lane * B
            for b in range(B):
                v = sb[b, pl.ds(go, 16)] * f
                plsc.store_scatter(ob, [base + b], v)
        pltpu.async_copy(ob, _tab_slice(k), wsem)
    pltpu.make_async_copy(obuf0 if (NGC - 2) % 2 == 0 else obuf1,
                          _tab_slice(NGC - 2), wsem).wait()
    pltpu.make_async_copy(obuf0 if (NGC - 1) % 2 == 0 else obuf1,
                          _tab_slice(NGC - 1), wsem).wait()


@functools.partial(
    pl.kernel,
    out_type=jax.ShapeDtypeStruct((NC * ACC_ROWS, B), jnp.float32),
    mesh=plsc.VectorSubcoreMesh(core_axis_name="c", subcore_axis_name="s"),
    scratch_types=[
        pltpu.VMEM((NCHUNK, CK), jnp.int32),
        pltpu.VMEM((NCHUNK, CK), jnp.int32),
        pltpu.VMEM((2, PH, CK, B), jnp.float32),
        pltpu.VMEM_SHARED((ACC_ROWS, B), jnp.float32),
        pltpu.SemaphoreType.DMA,
        pltpu.SemaphoreType.DMA,
    ],
    compiler_params=pltpu.CompilerParams(use_tc_tiling_on_sc=False),
)
def _sc_segsum(tab_hbm, ids_hbm, segs_hbm, out_hbm,
               idx_v, seg_v, rows_v, acc, gsem, ssem):
    c = lax.axis_index("c")
    s = lax.axis_index("s")
    wid = s * NC + c

    def _zrow(i, _):
        rows_v[0, 0, i, :] = jnp.zeros((B,), jnp.float32)
        return 0
    lax.fori_loop(0, CK, _zrow, 0)

    def _zacc(k, _):
        pltpu.sync_copy(rows_v.at[0, 0], acc.at[pl.ds(s * ZROWS + k * CK, CK)])
        return 0
    lax.fori_loop(0, ZROWS // CK, _zacc, 0)

    pltpu.sync_copy(ids_hbm.at[wid], idx_v)
    pltpu.sync_copy(segs_hbm.at[wid], seg_v)

    plsc.subcore_barrier()

    def _fire_g(ph, st):
        def f(r, _):
            pltpu.async_copy(tab_hbm.at[idx_v.at[ph * PH + r]],
                             rows_v.at[st, r], gsem)
            return 0
        lax.fori_loop(0, PH, f, 0)

    def _drain_g(ph, st):
        def f(r, _):
            pltpu.make_async_copy(tab_hbm.at[idx_v.at[ph * PH + r]],
                                  rows_v.at[st, r], gsem).wait()
            return 0
        lax.fori_loop(0, PH, f, 0)

    def _fire_s(ph, st):
        def f(r, _):
            pltpu.async_copy(rows_v.at[st, r],
                             acc.at[seg_v.at[ph * PH + r]], ssem, add=True)
            return 0
        lax.fori_loop(0, PH, f, 0)

    def _drain_s(ph, st):
        def f(r, _):
            pltpu.make_async_copy(rows_v.at[st, r],
                                  acc.at[seg_v.at[ph * PH + r]], ssem).wait()
            return 0
        lax.fori_loop(0, PH, f, 0)

    _fire_g(0, 0)
    for ph in range(NPH):
        st = ph % 2
        _drain_g(ph, st)
        if ph > 0:
            _drain_s(ph - 1, 1 - st)
        _fire_s(ph, st)
        if ph + 1 < NPH:
            _fire_g(ph + 1, 1 - st)
    _drain_s(NPH - 1, (NPH - 1) % 2)

    plsc.subcore_barrier()
    pltpu.sync_copy(acc.at[pl.ds(s * ZROWS, ZROWS)],
                    out_hbm.at[pl.ds(c * ACC_ROWS + s * ZROWS, ZROWS)])


_QROWS = ACC_ROWS // 8


def _mlp_body(parts_ref, W1r_ref, b1_ref, g1_ref, be1_ref,
              W2_ref, b2_ref, g2_ref, be2_ref, Wm_ref, bm_ref, out_ref):
    p = parts_ref[0:_QROWS, :] + parts_ref[_QROWS:2 * _QROWS, :]
    h = jnp.zeros((DH, B), jnp.float32)
    for j in range(8):
        h = h + lax.dot_general(W1r_ref[j], p[:, j * B:(j + 1) * B],
                                (((1,), (0,)), ((), ())),
                                preferred_element_type=jnp.float32)
    h = h + b1_ref[...]
    h = h * (g1_ref[...] * _BN_SCALE) + be1_ref[...]
    h = jnp.maximum(h, 0.0)
    h = lax.dot_general(W2_ref[...], h, (((1,), (0,)), ((), ())),
                        preferred_element_type=jnp.float32)
    h = h + b2_ref[...]
    h = h * (g2_ref[...] * _BN_SCALE) + be2_ref[...]
    h = jnp.maximum(h, 0.0)
    out_ref[...] = lax.dot_general(h, Wm_ref[...], (((0,), (1,)), ((), ())),
                                   preferred_element_type=jnp.float32) + bm_ref[...]


_mlp = pl.pallas_call(
    _mlp_body,
    out_shape=jax.ShapeDtypeStruct((B, 1), jnp.float32),
)


def kernel(snp, snp_ids, segment_ids, filters,
           W1, b1, bn1_w, bn1_b, W2, b2, bn2_w, bn2_b, Wm, bm):
    pad = NPAD - N_NODES
    pad_pos = np.arange(pad, dtype=np.int32)
    pad_idx = jnp.asarray(pad_pos % N_SNPS)
    pad_seg = jnp.asarray(N_GENES + (pad_pos % (ACC_ROWS - N_GENES)))
    ids3 = jnp.concatenate([snp_ids, pad_idx]).reshape(NW, NCHUNK, CK)
    segs3 = jnp.concatenate([segment_ids, pad_seg]).reshape(NW, NCHUNK, CK)
    Wm_p = jnp.pad(Wm, ((0, 0), (0, FD - MAIN_DIM)))
    W1r = jnp.pad(W1.reshape(DH, N_GENES // 8, 8).transpose(2, 0, 1),
                  ((0, 0), (0, 0), (0, (ACC_ROWS - N_GENES) // 8)))
    tab_flat = _sc_build(snp, filters)
    tab = tab_flat.reshape(SPAD, B)
    parts = _sc_segsum(tab, ids3, segs3)
    p2 = parts.reshape(NC * ACC_ROWS * B // 128, 128)
    return _mlp(p2, W1r,
                b1.reshape(DH, 1), bn1_w.reshape(DH, 1), bn1_b.reshape(DH, 1),
                W2, b2.reshape(FD, 1), bn2_w.reshape(FD, 1), bn2_b.reshape(FD, 1),
                Wm_p, bm.reshape(1, 1))
